# baseline (speedup 1.0000x reference)
import functools

import jax
import jax.numpy as jnp
from jax import lax
from jax.experimental import pallas as pl
from jax.experimental.pallas import tpu as pltpu
from jax.experimental.pallas import tpu_sc as plsc

BATCH = 4096
MAXLEN = 200
EMBED = 64
LANES = 16

_info = plsc.get_sparse_core_info()
NC, NS = _info.num_cores, _info.num_subcores
NW = NC * NS
BPW = BATCH // NW
VPR = EMBED // LANES


def _body(x_hbm, pos_hbm, tok_hbm, out_hbm,
          idx_v, pos_v, idxc0, idxc1, rows0, rows1, g0, g1):
    w = lax.axis_index("s") * NC + lax.axis_index("c")
    b0 = w * BPW
    pltpu.sync_copy(x_hbm.at[pl.ds(b0, BPW)], idx_v)
    pltpu.sync_copy(pos_hbm, pos_v)
    rows = (rows0, rows1)
    idxc = (idxc0, idxc1)
    sems = (g0, g1)

    def transpose_col(t, b):
        dst = idxc[b]
        cols = jnp.full((LANES,), t, jnp.int32)
        for v in range(BPW // LANES):
            rws = lax.iota(jnp.int32, LANES) + (v * LANES)
            dst[pl.ds(v * LANES, LANES)] = plsc.load_gather(idx_v, [rws, cols])

    def gather_start(b):
        pltpu.make_async_copy(tok_hbm.at[idxc[b]], rows[b], sems[b]).start()

    def gather_wait(b):
        pltpu.make_async_copy(tok_hbm.at[idxc[b]], rows[b], sems[b]).wait()

    def add_pos(t, b):
        rbuf = rows[b]
        pv = [pos_v[t, pl.ds(k * LANES, LANES)] for k in range(VPR)]

        def row(r, carry):
            for k in range(VPR):
                sl = pl.ds(k * LANES, LANES)
                rbuf[r, sl] = rbuf[r, sl] + pv[k]
            return carry

        lax.fori_loop(0, BPW, row, 0, unroll=4)

    def store(t, b):
        pltpu.sync_copy(rows[b], out_hbm.at[pl.ds(b0, BPW), t])

    transpose_col(0, 0)
    gather_start(0)

    def outer(i, carry):
        t0 = i * 2
        transpose_col(t0 + 1, 1)
        gather_start(1)
        gather_wait(0)
        add_pos(t0, 0)
        store(t0, 0)

        @pl.when(t0 + 2 < MAXLEN)
        def _():
            transpose_col(t0 + 2, 0)
            gather_start(0)

        gather_wait(1)
        add_pos(t0 + 1, 1)
        store(t0 + 1, 1)
        return carry

    lax.fori_loop(0, MAXLEN // 2, outer, 0)


@jax.jit
def kernel(x, token_table, pos_table):
    B, T = x.shape
    V, D = token_table.shape
    assert (B, T, D) == (BATCH, MAXLEN, EMBED)

    run = pl.kernel(
        _body,
        out_type=jax.ShapeDtypeStruct((B, T, D), jnp.float32),
        mesh=plsc.VectorSubcoreMesh(core_axis_name="c", subcore_axis_name="s"),
        compiler_params=pltpu.CompilerParams(
            use_tc_tiling_on_sc=False, needs_layout_passes=False
        ),
        scratch_types=[
            pltpu.VMEM((BPW, T), jnp.int32),
            pltpu.VMEM((T, EMBED), jnp.float32),
            pltpu.VMEM((BPW,), jnp.int32),
            pltpu.VMEM((BPW,), jnp.int32),
            pltpu.VMEM((BPW, EMBED), jnp.float32),
            pltpu.VMEM((BPW, EMBED), jnp.float32),
            pltpu.SemaphoreType.DMA,
            pltpu.SemaphoreType.DMA,
        ],
    )
    return run(x.astype(jnp.int32), pos_table, token_table)
